# Initial kernel scaffold; baseline (speedup 1.0000x reference)
import jax
import jax.numpy as jnp
from jax.experimental import pallas as pl


def kernel(x, edge_index, batch, emb_table, Wrel1, brel1, Wroot1, Wrel2, brel2, Wroot2, Wrel3, brel3, Wroot3, lin1_W, lin1_b, lin2_W, lin2_b, lin3_W, lin3_b, bn1_g, bn1_b, bn2_g, bn2_b):
    raise NotImplementedError("write your pallas kernel here")


# speedup vs baseline: 3.2220x; 3.2220x over previous
import functools

import jax
import jax.numpy as jnp
from jax import lax
from jax.experimental import pallas as pl
from jax.experimental.pallas import tpu as pltpu
from jax.experimental.pallas import tpu_sc as plsc

N = 10000
E = 320000
D = 128
B = 256

NC = 2
NS = 16
NW = NC * NS

N_PAD = 10240
ROWS_PT = N_PAD // NW
ACC_PT = N_PAD // NS
EC = 128
EPT_CHUNKS = 79
EPT = EPT_CHUNKS * EC
E_PAD = EPT * NW
GC = 64
PAD_GRAPH = 300

_mesh = plsc.VectorSubcoreMesh(
    core_axis_name="c", subcore_axis_name="s", num_cores=NC, num_subcores=NS)


def _zero_rows(zbuf, nrows):
  def zrow(r, carry):
    for cc in range(D // 16):
      zbuf[r, pl.ds(cc * 16, 16)] = jnp.zeros((16,), jnp.float32)
    return carry
  lax.fori_loop(0, nrows, zrow, 0)


@functools.partial(
    pl.kernel,
    out_type=jax.ShapeDtypeStruct((N_PAD, D), jnp.float32),
    mesh=_mesh,
    scratch_types=[
        pltpu.VMEM((GC,), jnp.int32),
        pltpu.VMEM((GC, D), jnp.float32),
        pltpu.SemaphoreType.DMA,
    ],
)
def _emb_kernel(table_hbm, idx_hbm, out_hbm, idx_v, rows_v, sem):
  wid = lax.axis_index("s") * NC + lax.axis_index("c")
  base = wid * ROWS_PT
  for j in range(ROWS_PT // GC):
    off = base + j * GC
    pltpu.sync_copy(idx_hbm.at[pl.ds(off, GC)], idx_v)
    pltpu.async_copy(table_hbm.at[idx_v], rows_v, sem).wait()
    pltpu.sync_copy(rows_v, out_hbm.at[pl.ds(off, GC)])


@functools.partial(
    pl.kernel,
    out_type=jax.ShapeDtypeStruct((NC, N_PAD, D), jnp.float32),
    mesh=_mesh,
    scratch_types=[
        pltpu.VMEM((EC,), jnp.int32),
        pltpu.VMEM((EC,), jnp.int32),
        pltpu.VMEM((EC, D), jnp.float32),
        pltpu.VMEM((GC, D), jnp.float32),
        pltpu.VMEM_SHARED((N_PAD, D), jnp.float32),
        pltpu.SemaphoreType.DMA,
    ],
)
def _conv_kernel(h_hbm, src_hbm, dst_hbm, out_hbm,
                 idx_s, idx_d, rows_v, zbuf, acc, sem):
  c = lax.axis_index("c")
  s = lax.axis_index("s")
  wid = s * NC + c

  _zero_rows(zbuf, GC)
  for j in range(ACC_PT // GC):
    pltpu.sync_copy(zbuf, acc.at[pl.ds(s * ACC_PT + j * GC, GC)])
  plsc.subcore_barrier()

  ebase = wid * EPT

  def body(ch, carry):
    off = ebase + ch * EC
    pltpu.sync_copy(src_hbm.at[pl.ds(off, EC)], idx_s)
    pltpu.sync_copy(dst_hbm.at[pl.ds(off, EC)], idx_d)
    pltpu.async_copy(h_hbm.at[idx_s], rows_v, sem).wait()
    pltpu.sync_copy(rows_v, acc.at[idx_d], add=True)
    return carry

  lax.fori_loop(0, EPT_CHUNKS, body, 0)
  plsc.subcore_barrier()

  for j in range(ACC_PT // GC):
    off = s * ACC_PT + j * GC
    pltpu.sync_copy(acc.at[pl.ds(off, GC)], out_hbm.at[c, pl.ds(off, GC)])


_BLK = 256
_DN = (((1,), (1,)), ((), ()))


def _conv_tc_body(agg0_ref, agg1_ref, h_ref, wrel_ref, brel_ref, wroot_ref,
                  out_ref):
  a = agg0_ref[...] + agg1_ref[...]
  t = lax.dot_general(a, wrel_ref[...], _DN,
                      preferred_element_type=jnp.float32,
                      precision=lax.Precision.DEFAULT)
  t = t + brel_ref[...]
  t = t + lax.dot_general(h_ref[...], wroot_ref[...], _DN,
                          preferred_element_type=jnp.float32,
                          precision=lax.Precision.DEFAULT)
  out_ref[...] = jnp.where(t > 0, t, 0.01 * t)


def _conv_tc(agg0, agg1, h, wrel, brel, wroot):
  blk = pl.BlockSpec((_BLK, D), lambda i: (i, 0))
  full = pl.BlockSpec((D, D), lambda i: (0, 0))
  return pl.pallas_call(
      _conv_tc_body,
      grid=(N_PAD // _BLK,),
      in_specs=[blk, blk, blk, full, pl.BlockSpec((1, D), lambda i: (0, 0)),
                full],
      out_specs=blk,
      out_shape=jax.ShapeDtypeStruct((N_PAD, D), jnp.float32),
  )(agg0, agg1, h, wrel, brel, wroot)


def _head_body(h_ref, bidx_ref, l1w_ref, l1b_ref, l2w_ref, l2b_ref,
               l3w_ref, l3b_ref, g1_ref, b1_ref, g2_ref, b2_ref, out_ref):
  h = h_ref[...]
  bidx = bidx_ref[...]
  iota = lax.broadcasted_iota(jnp.int32, (1, B), 1)
  onehot = (bidx == iota).astype(jnp.float32)
  counts = jnp.maximum(jnp.sum(onehot, axis=0), 1.0)
  pooled = lax.dot_general(onehot, h, (((0,), (0,)), ((), ())),
                           preferred_element_type=jnp.float32,
                           precision=lax.Precision.DEFAULT)
  g = pooled / counts[:, None]

  def dense(v, w, b):
    return lax.dot_general(v, w, _DN, preferred_element_type=jnp.float32,
                           precision=lax.Precision.DEFAULT) + b

  def bn(v, gamma, beta):
    mu = jnp.mean(v, axis=0, keepdims=True)
    var = jnp.mean((v - mu) * (v - mu), axis=0, keepdims=True)
    return (v - mu) * lax.rsqrt(var + 1e-5) * gamma + beta

  t = bn(dense(g, l1w_ref[...], l1b_ref[...]), g1_ref[...], b1_ref[...])
  t = jnp.where(t > 0, t, 0.01 * t)
  t = bn(dense(t, l2w_ref[...], l2b_ref[...]), g2_ref[...], b2_ref[...])
  t = jnp.where(t > 0, t, 0.01 * t)
  out_ref[...] = (jnp.sum(t * l3w_ref[...], axis=1, keepdims=True)
                  + l3b_ref[...])


def _head(h3, bidx, l1w, l1b, l2w, l2b, l3w, l3b, g1, b1, g2, b2):
  full = lambda shape: pl.BlockSpec(shape, lambda: (0,) * len(shape))
  args = [h3, bidx, l1w, l1b, l2w, l2b, l3w, l3b, g1, b1, g2, b2]
  return pl.pallas_call(
      _head_body,
      in_specs=[full(a.shape) for a in args],
      out_specs=full((B, 1)),
      out_shape=jax.ShapeDtypeStruct((B, 1), jnp.float32),
  )(*args)


def kernel(x, edge_index, batch, emb_table, Wrel1, brel1, Wroot1, Wrel2,
           brel2, Wroot2, Wrel3, brel3, Wroot3, lin1_W, lin1_b, lin2_W,
           lin2_b, lin3_W, lin3_b, bn1_g, bn1_b, bn2_g, bn2_b):
  x = x.astype(jnp.int32)
  x_pad = jnp.concatenate(
      [x, jnp.zeros((N_PAD - N,), jnp.int32)])
  src = edge_index[0].astype(jnp.int32)
  dst = edge_index[1].astype(jnp.int32)
  epad = E_PAD - E
  src = jnp.concatenate([src, jnp.zeros((epad,), jnp.int32)])
  dst = jnp.concatenate([dst, jnp.full((epad,), N, jnp.int32)])
  bidx = jnp.concatenate(
      [batch.astype(jnp.int32),
       jnp.full((N_PAD - N,), PAD_GRAPH, jnp.int32)]).reshape(N_PAD, 1)

  h = _emb_kernel(emb_table, x_pad)

  for wrel, brel, wroot in ((Wrel1, brel1, Wroot1),
                            (Wrel2, brel2, Wroot2),
                            (Wrel3, brel3, Wroot3)):
    aggs = _conv_kernel(h, src, dst)
    h = _conv_tc(aggs[0], aggs[1], h, wrel, brel.reshape(1, D), wroot)

  out = _head(h, bidx,
              lin1_W, lin1_b.reshape(1, 1024),
              lin2_W, lin2_b.reshape(1, D),
              lin3_W, lin3_b.reshape(1, 1),
              bn1_g.reshape(1, 1024), bn1_b.reshape(1, 1024),
              bn2_g.reshape(1, D), bn2_b.reshape(1, D))
  return out[:, 0]
